# Initial kernel scaffold; baseline (speedup 1.0000x reference)
import jax
import jax.numpy as jnp
from jax.experimental import pallas as pl


def kernel(x, r_ij, neighbors, pairwise_mask, fsblock_ij, fpblock_ij, Wf1_s, bf1_s, Wf2_s, bf2_s, Wf1_p, bf1_p, Wf2_p, bf2_p, W_s, W_p, W_out, b_out):
    raise NotImplementedError("write your pallas kernel here")


# speedup vs baseline: 4.8245x; 4.8245x over previous
import jax
import jax.numpy as jnp
from jax.experimental import pallas as pl
from jax.experimental.pallas import tpu as pltpu
from jax.experimental.pallas import tpu_sc as plsc

Nb, Na, Nnbh = 2, 5000, 16
NIN, NF, NOUT, NG = 128, 128, 128, 64

A_BLOCK = 200
R_BLOCK = A_BLOCK * Nnbh
GATHER_WINDOW = 256
SC_UNITS = 32
_LN2 = 0.6931471805599453
_LOG2E = 1.4426950408889634


def _softplus(v):
    t = jnp.exp2(jnp.abs(v) * (-_LOG2E))
    return jnp.maximum(v, 0.0) + jnp.log2(1.0 + t) * _LN2


def _sc_gather(table, idx_pad):
    n_idx = idx_pad.shape[0]
    c = table.shape[1]
    idx2 = idx_pad.reshape(1, n_idx)
    mesh = plsc.VectorSubcoreMesh(core_axis_name="c", subcore_axis_name="s")

    @pl.kernel(out_type=jax.ShapeDtypeStruct((n_idx, c), table.dtype), mesh=mesh)
    def gather_kernel(x_hbm, i_hbm, o_hbm):
        def body(i_vmem, o_vmem):
            pltpu.sync_copy(x_hbm.at[i_vmem.at[0]], o_vmem)

        pltpu.emit_pipeline(
            body,
            grid=(n_idx // GATHER_WINDOW,),
            in_specs=[pl.BlockSpec((1, GATHER_WINDOW), index_map=lambda i: (0, i))],
            out_specs=[pl.BlockSpec((GATHER_WINDOW, c), index_map=lambda i: (i, 0))],
            core_axis_name=("c", "s"),
            dimension_semantics=(pltpu.PARALLEL,),
        )(i_hbm, o_hbm)

    return gather_kernel(table, idx2)


def _fused_body(xg_ref, mask_ref, fs_ref, fp0_ref, fp1_ref, fp2_ref,
                wf1s_ref, bf1s_ref, wf2s_ref, bf2s_ref,
                wf1p_ref, bf1p_ref, wf2p_ref, bf2p_ref,
                ws_ref, wp_ref, wout_ref, bout_ref, o_ref):
    f32 = jnp.float32
    xg = xg_ref[...]
    mask3 = mask_ref[...][:, :, None]

    gs = jnp.dot(xg, ws_ref[...], preferred_element_type=f32)
    gp = jnp.dot(xg, wp_ref[...], preferred_element_type=f32)
    gs3 = gs.reshape(A_BLOCK, Nnbh, NF) * mask3
    gp3 = gp.reshape(A_BLOCK, Nnbh, NF) * mask3

    hs = _softplus(jnp.dot(fs_ref[...], wf1s_ref[...], preferred_element_type=f32)
                   + bf1s_ref[...])
    Hs = jnp.dot(hs, wf2s_ref[...], preferred_element_type=f32) + bf2s_ref[...]
    ys = jnp.sum(gs3 * Hs.reshape(A_BLOCK, Nnbh, NF), axis=1)

    yp = jnp.zeros((A_BLOCK, NF), f32)
    for fpk_ref in (fp0_ref, fp1_ref, fp2_ref):
        hk = _softplus(jnp.dot(fpk_ref[...], wf1p_ref[...], preferred_element_type=f32)
                       + bf1p_ref[...])
        Hk = jnp.dot(hk, wf2p_ref[...], preferred_element_type=f32) + bf2p_ref[...]
        Sk = jnp.sum(gp3 * Hk.reshape(A_BLOCK, Nnbh, NF), axis=1)
        yp = yp + Sk * Sk

    y = ys + yp
    o_ref[...] = jnp.dot(y, wout_ref[...], preferred_element_type=f32) + bout_ref[...]


def _fused_specs():
    def full(shape):
        return pl.BlockSpec(shape, lambda i: (0,) * len(shape))

    in_specs = [
        pl.BlockSpec((R_BLOCK, NIN), lambda i: (i, 0)),
        pl.BlockSpec((A_BLOCK, Nnbh), lambda i: (i, 0)),
        pl.BlockSpec((R_BLOCK, NG), lambda i: (i, 0)),
        pl.BlockSpec((R_BLOCK, NG), lambda i: (i, 0)),
        pl.BlockSpec((R_BLOCK, NG), lambda i: (i, 0)),
        pl.BlockSpec((R_BLOCK, NG), lambda i: (i, 0)),
        full((NG, NF)), full((1, NF)), full((NF, NF)), full((1, NF)),
        full((NG, NF)), full((1, NF)), full((NF, NF)), full((1, NF)),
        full((NIN, NF)), full((NIN, NF)), full((NF, NOUT)), full((1, NOUT)),
    ]
    out_spec = pl.BlockSpec((A_BLOCK, NOUT), lambda i: (i, 0))
    grid = (Na // A_BLOCK,)
    return grid, in_specs, out_spec


def kernel(x, r_ij, neighbors, pairwise_mask, fsblock_ij, fpblock_ij,
           Wf1_s, bf1_s, Wf2_s, bf2_s, Wf1_p, bf1_p, Wf2_p, bf2_p,
           W_s, W_p, W_out, b_out):
    bf2_s_eff = (bf2_s - _LN2 * jnp.sum(Wf2_s, axis=0)).reshape(1, NF)
    bf2_p_eff = (bf2_p - _LN2 * jnp.sum(Wf2_p, axis=0)).reshape(1, NF)

    grid, in_specs, out_spec = _fused_specs()
    pad = (-(Na * Nnbh)) % (GATHER_WINDOW * SC_UNITS)
    zpad = jnp.zeros((pad,), jnp.int32)

    outs = []
    for b in range(Nb):
        idx_b = jnp.concatenate([neighbors[b].reshape(-1), zpad])
        xg_b = _sc_gather(x[b], idx_b)
        mask_b = pairwise_mask[b]
        fs_b = fsblock_ij[b].reshape(Na * Nnbh, NG)
        fp_b = [fpblock_ij[b, :, :, k, :].reshape(Na * Nnbh, NG) for k in range(3)]
        y_b = pl.pallas_call(
            _fused_body,
            grid=grid,
            in_specs=in_specs,
            out_specs=out_spec,
            out_shape=jax.ShapeDtypeStruct((Na, NOUT), jnp.float32),
        )(xg_b, mask_b, fs_b, fp_b[0], fp_b[1], fp_b[2],
          Wf1_s, bf1_s.reshape(1, NF), Wf2_s, bf2_s_eff,
          Wf1_p, bf1_p.reshape(1, NF), Wf2_p, bf2_p_eff,
          W_s, W_p, W_out, b_out.reshape(1, NOUT))
        outs.append(y_b)
    return jnp.stack(outs)
